# Initial kernel scaffold; baseline (speedup 1.0000x reference)
import functools

import jax
import jax.numpy as jnp
from jax import lax
from jax.experimental import pallas as pl
from jax.experimental.pallas import tpu as pltpu
from jax.experimental.pallas import tpu_sc as plsc

N = 10000
E = 320000
D = 128
H = 8
C = 16
HC = H * C
AW = 16
ACCW = HC + AW

NC = 2
NS = 16
TILE_E = E // (NC * NS)
CHUNK = 80
NPT = N // NS


RB = 1000


def _proj_body(x_ref, w_ref, as_ref, ad_ref, h_ref, aso_ref, ado_ref):
    h = jnp.dot(x_ref[...], w_ref[...], preferred_element_type=jnp.float32)
    h_ref[...] = h
    aso_ref[...] = jnp.dot(h, as_ref[...], preferred_element_type=jnp.float32)
    ado_ref[...] = jnp.dot(h, ad_ref[...], preferred_element_type=jnp.float32)


def _proj(x, w, a_s, a_d):
    return pl.pallas_call(
        _proj_body,
        grid=(N // RB,),
        in_specs=[
            pl.BlockSpec((RB, D), lambda i: (i, 0)),
            pl.BlockSpec((D, HC), lambda i: (0, 0)),
            pl.BlockSpec((HC, AW), lambda i: (0, 0)),
            pl.BlockSpec((HC, AW), lambda i: (0, 0)),
        ],
        out_specs=[
            pl.BlockSpec((RB, HC), lambda i: (i, 0)),
            pl.BlockSpec((RB, AW), lambda i: (i, 0)),
            pl.BlockSpec((RB, AW), lambda i: (i, 0)),
        ],
        out_shape=[
            jax.ShapeDtypeStruct((N, HC), jnp.float32),
            jax.ShapeDtypeStruct((N, AW), jnp.float32),
            jax.ShapeDtypeStruct((N, AW), jnp.float32),
        ],
    )(x, w, a_s, a_d)


def _combine_body(acc_ref, b_ref, krep_ref, o_ref, *, act):
    num = acc_ref[0, :, 0:HC] + acc_ref[1, :, 0:HC]
    den = acc_ref[0, :, HC:ACCW] + acc_ref[1, :, HC:ACCW]
    den_rep = jnp.dot(den, krep_ref[...], preferred_element_type=jnp.float32)
    out = num / (den_rep + 1e-16) + b_ref[...]
    if act:
        out = jnp.where(out > 0.0, out, jnp.expm1(out))
    o_ref[...] = out


def _combine(acc, b, krep, act):
    return pl.pallas_call(
        functools.partial(_combine_body, act=act),
        grid=(N // RB,),
        in_specs=[
            pl.BlockSpec((NC, RB, ACCW), lambda i: (0, i, 0)),
            pl.BlockSpec((1, HC), lambda i: (0, 0)),
            pl.BlockSpec((AW, HC), lambda i: (0, 0)),
        ],
        out_specs=pl.BlockSpec((RB, HC), lambda i: (i, 0)),
        out_shape=jax.ShapeDtypeStruct((N, HC), jnp.float32),
    )(acc, b, krep)


_MESH = plsc.VectorSubcoreMesh(core_axis_name="c", subcore_axis_name="s")


@functools.partial(
    pl.kernel,
    out_type=jax.ShapeDtypeStruct((NC, N, ACCW), jnp.float32),
    mesh=_MESH,
    scratch_types=[
        pltpu.VMEM((CHUNK,), jnp.int32),
        pltpu.VMEM((CHUNK,), jnp.int32),
        pltpu.VMEM((CHUNK, HC), jnp.float32),
        pltpu.VMEM((CHUNK, AW), jnp.float32),
        pltpu.VMEM((CHUNK, AW), jnp.float32),
        pltpu.VMEM((CHUNK, ACCW), jnp.float32),
        pltpu.VMEM((16,), jnp.float32),
        pltpu.VMEM_SHARED((N, ACCW), jnp.float32),
        pltpu.SemaphoreType.DMA,
        pltpu.SemaphoreType.DMA,
        pltpu.SemaphoreType.DMA,
    ],
)
def _edge_kernel(h_hbm, as_hbm, ad_hbm, src_hbm, dst_hbm, zeros_hbm, out_hbm,
                 sidx, didx, hbuf, asbuf, adbuf, sbuf, zscr, acc,
                 sem1, sem2, sem3):
    c = lax.axis_index("c")
    s = lax.axis_index("s")

    pltpu.sync_copy(zeros_hbm.at[pl.ds(s * NPT, NPT)],
                    acc.at[pl.ds(s * NPT, NPT)])
    plsc.subcore_barrier()

    base_edge = (c * NS + s) * TILE_E
    lanes = lax.iota(jnp.int32, 16)

    def chunk_body(k, carry):
        eb = base_edge + k * CHUNK
        pltpu.sync_copy(src_hbm.at[pl.ds(eb, CHUNK)], sidx)
        pltpu.sync_copy(dst_hbm.at[pl.ds(eb, CHUNK)], didx)
        cp1 = pltpu.async_copy(h_hbm.at[sidx], hbuf, sem1)
        cp2 = pltpu.async_copy(as_hbm.at[sidx], asbuf, sem2)
        cp3 = pltpu.async_copy(ad_hbm.at[didx], adbuf, sem3)
        cp1.wait()
        cp2.wait()
        cp3.wait()

        def edge_body(i, carry2):
            e = asbuf[i, :] + adbuf[i, :]
            e = jnp.maximum(e, e * 0.2)
            z = jnp.exp(e)
            z = jnp.where(lanes < H, z, 0.0)
            sbuf[i, pl.ds(HC, AW)] = z
            zscr[...] = z
            for j in range(H):
                zj = plsc.load_gather(
                    zscr, [jnp.full((16,), j, jnp.int32)])
                sbuf[i, pl.ds(j * C, C)] = hbuf[i, pl.ds(j * C, C)] * zj
            return carry2

        lax.fori_loop(0, CHUNK, edge_body, 0)
        pltpu.sync_copy(sbuf, acc.at[didx], add=True)
        return carry

    lax.fori_loop(0, TILE_E // CHUNK, chunk_body, 0)
    plsc.subcore_barrier()
    pltpu.sync_copy(acc.at[pl.ds(s * NPT, NPT)],
                    out_hbm.at[c, pl.ds(s * NPT, NPT)])


def _att_mat(a):
    eye = jnp.eye(H, AW, dtype=jnp.float32)
    return (a[:, :, None] * eye[:, None, :]).reshape(HC, AW)


def kernel(x, edge_index, W1, a_src1, a_dst1, b1, W2, a_src2, a_dst2, b2):
    src = edge_index[0]
    dst = edge_index[1]
    zeros = jnp.zeros((N, ACCW), jnp.float32)
    krep = jnp.repeat(jnp.eye(AW, H, dtype=jnp.float32), C, axis=1)

    h1, as1, ad1 = _proj(x, W1, _att_mat(a_src1), _att_mat(a_dst1))
    acc1 = _edge_kernel(h1, as1, ad1, src, dst, zeros)
    g1 = _combine(acc1, b1.reshape(1, HC), krep, act=True)

    h2, as2, ad2 = _proj(g1, W2, _att_mat(a_src2), _att_mat(a_dst2))
    acc2 = _edge_kernel(h2, as2, ad2, src, dst, zeros)
    return _combine(acc2, b2.reshape(1, HC), krep, act=False)


# speedup vs baseline: 44.6917x; 44.6917x over previous
import functools

import jax
import jax.numpy as jnp
from jax import lax
from jax.experimental import pallas as pl
from jax.experimental.pallas import tpu as pltpu
from jax.experimental.pallas import tpu_sc as plsc

N = 10000
E = 320000
D = 128
H = 8
C = 16
HC = H * C
AW = 16
ACCW = HC + AW

NC = 2
NS = 16
CHUNK = 80


def _make_proj(n, rb, interpret=False):
    def body(x_ref, w_ref, as_ref, ad_ref, h_ref, aso_ref, ado_ref):
        h = jnp.dot(x_ref[...], w_ref[...], preferred_element_type=jnp.float32)
        h_ref[...] = h
        aso_ref[...] = jnp.dot(h, as_ref[...], preferred_element_type=jnp.float32)
        ado_ref[...] = jnp.dot(h, ad_ref[...], preferred_element_type=jnp.float32)

    return pl.pallas_call(
        body,
        grid=(n // rb,),
        in_specs=[
            pl.BlockSpec((rb, D), lambda i: (i, 0)),
            pl.BlockSpec((D, HC), lambda i: (0, 0)),
            pl.BlockSpec((HC, AW), lambda i: (0, 0)),
            pl.BlockSpec((HC, AW), lambda i: (0, 0)),
        ],
        out_specs=[
            pl.BlockSpec((rb, HC), lambda i: (i, 0)),
            pl.BlockSpec((rb, AW), lambda i: (i, 0)),
            pl.BlockSpec((rb, AW), lambda i: (i, 0)),
        ],
        out_shape=[
            jax.ShapeDtypeStruct((n, HC), jnp.float32),
            jax.ShapeDtypeStruct((n, AW), jnp.float32),
            jax.ShapeDtypeStruct((n, AW), jnp.float32),
        ],
        interpret=interpret,
    )


def _make_combine(n, rb, act, interpret=False):
    def body(acc_ref, b_ref, krep_ref, o_ref):
        num = acc_ref[0, :, 0:HC] + acc_ref[1, :, 0:HC]
        den = acc_ref[0, :, HC:ACCW] + acc_ref[1, :, HC:ACCW]
        den_rep = jnp.dot(den, krep_ref[...], preferred_element_type=jnp.float32)
        out = num / (den_rep + 1e-16) + b_ref[...]
        if act:
            out = jnp.where(out > 0.0, out, jnp.exp(jnp.minimum(out, 0.0)) - 1.0)
        o_ref[...] = out

    return pl.pallas_call(
        body,
        grid=(n // rb,),
        in_specs=[
            pl.BlockSpec((NC, rb, ACCW), lambda i: (0, i, 0)),
            pl.BlockSpec((1, HC), lambda i: (0, 0)),
            pl.BlockSpec((AW, HC), lambda i: (0, 0)),
        ],
        out_specs=pl.BlockSpec((rb, HC), lambda i: (i, 0)),
        out_shape=jax.ShapeDtypeStruct((n, HC), jnp.float32),
        interpret=interpret,
    )


def _bcast_lane(v, j):
    dn = lax.GatherDimensionNumbers(
        offset_dims=(), collapsed_slice_dims=(0,), start_index_map=(0,))
    return lax.gather(
        v, jnp.full((16, 1), j, jnp.int32), dn, (1,),
        mode=lax.GatherScatterMode.PROMISE_IN_BOUNDS)


def _make_edge_kernel(n, e, chunk, interpret=False):
    tile_e = e // (NC * NS)
    npt = n // NS
    mesh = plsc.VectorSubcoreMesh(core_axis_name="c", subcore_axis_name="s")

    @functools.partial(
        pl.kernel,
        out_type=jax.ShapeDtypeStruct((NC, n, ACCW), jnp.float32),
        mesh=mesh,
        compiler_params=pltpu.CompilerParams(
            use_tc_tiling_on_sc=False, needs_layout_passes=False),
        scratch_types=[
            pltpu.VMEM((chunk,), jnp.int32),
            pltpu.VMEM((chunk,), jnp.int32),
            pltpu.VMEM((chunk, HC), jnp.float32),
            pltpu.VMEM((chunk, AW), jnp.float32),
            pltpu.VMEM((chunk, AW), jnp.float32),
            pltpu.VMEM((chunk, ACCW), jnp.float32),
            pltpu.VMEM_SHARED((n, ACCW), jnp.float32),
            pltpu.SemaphoreType.DMA,
            pltpu.SemaphoreType.DMA,
            pltpu.SemaphoreType.DMA,
        ],
        interpret=interpret,
    )
    def edge_kernel(h_hbm, as_hbm, ad_hbm, src_hbm, dst_hbm, zeros_hbm,
                    out_hbm, sidx, didx, hbuf, asbuf, adbuf, sbuf, acc,
                    sem1, sem2, sem3):
        c = lax.axis_index("c")
        s = lax.axis_index("s")

        pltpu.sync_copy(zeros_hbm.at[pl.ds(s * npt, npt)],
                        acc.at[pl.ds(s * npt, npt)])
        plsc.subcore_barrier()

        base_edge = (c * NS + s) * tile_e
        lanes = lax.iota(jnp.int32, 16)

        def chunk_body(k, carry):
            eb = base_edge + k * chunk
            pltpu.sync_copy(src_hbm.at[pl.ds(eb, chunk)], sidx)
            pltpu.sync_copy(dst_hbm.at[pl.ds(eb, chunk)], didx)
            cp1 = pltpu.async_copy(h_hbm.at[sidx], hbuf, sem1)
            cp2 = pltpu.async_copy(as_hbm.at[sidx], asbuf, sem2)
            cp3 = pltpu.async_copy(ad_hbm.at[didx], adbuf, sem3)
            cp1.wait()
            cp2.wait()
            cp3.wait()

            def edge_body(i, carry2):
                ev = asbuf[i, :] + adbuf[i, :]
                ev = jnp.maximum(ev, ev * 0.2)
                z = jnp.exp(ev)
                z = jnp.where(lanes < H, z, 0.0)
                sbuf[i, pl.ds(HC, AW)] = z
                for j in range(H):
                    zj = _bcast_lane(z, j)
                    sbuf[i, pl.ds(j * C, C)] = hbuf[i, pl.ds(j * C, C)] * zj
                return carry2

            lax.fori_loop(0, chunk, edge_body, 0)
            pltpu.sync_copy(sbuf, acc.at[didx], add=True)
            return carry

        lax.fori_loop(0, tile_e // chunk, chunk_body, 0)
        plsc.subcore_barrier()
        pltpu.sync_copy(acc.at[pl.ds(s * npt, npt)],
                        out_hbm.at[c, pl.ds(s * npt, npt)])

    return edge_kernel


def _att_mat(a):
    eye = jnp.eye(H, AW, dtype=jnp.float32)
    return (a[:, :, None] * eye[:, None, :]).reshape(HC, AW)


_PROJ = _make_proj(N, 1000)
_EDGE = _make_edge_kernel(N, E, CHUNK)
_COMBINE_ELU = _make_combine(N, 1000, act=True)
_COMBINE = _make_combine(N, 1000, act=False)


def kernel(x, edge_index, W1, a_src1, a_dst1, b1, W2, a_src2, a_dst2, b2):
    src = edge_index[0]
    dst = edge_index[1]
    zeros = jnp.zeros((N, ACCW), jnp.float32)
    krep = jnp.repeat(jnp.eye(AW, H, dtype=jnp.float32), C, axis=1)

    h1, as1, ad1 = _PROJ(x, W1, _att_mat(a_src1), _att_mat(a_dst1))
    acc1 = _EDGE(h1, as1, ad1, src, dst, zeros)
    g1 = _COMBINE_ELU(acc1, b1.reshape(1, HC), krep)

    h2, as2, ad2 = _PROJ(g1, W2, _att_mat(a_src2), _att_mat(a_dst2))
    acc2 = _EDGE(h2, as2, ad2, src, dst, zeros)
    return _COMBINE(acc2, b2.reshape(1, HC), krep)


# speedup vs baseline: 145.1891x; 3.2487x over previous
import functools

import jax
import jax.numpy as jnp
from jax import lax
from jax.experimental import pallas as pl
from jax.experimental.pallas import tpu as pltpu
from jax.experimental.pallas import tpu_sc as plsc

N = 10000
E = 320000
D = 128
H = 8
C = 16
HC = H * C
AW = 16
PW = HC + AW

NC = 2
NS = 16
CHUNK = 40


def _make_proj(n, rb, interpret=False):
    def body(x_ref, w_ref, as_ref, ad_ref, adb_ref, p_ref, ado_ref):
        h = jnp.dot(x_ref[...], w_ref[...], preferred_element_type=jnp.float32)
        asv = jnp.dot(h, as_ref[...], preferred_element_type=jnp.float32)
        p_ref[...] = jnp.concatenate([h, asv], axis=1)
        ado_ref[...] = (
            jnp.dot(h, ad_ref[...], preferred_element_type=jnp.float32)
            + adb_ref[...])

    return pl.pallas_call(
        body,
        grid=(n // rb,),
        in_specs=[
            pl.BlockSpec((rb, D), lambda i: (i, 0)),
            pl.BlockSpec((D, HC), lambda i: (0, 0)),
            pl.BlockSpec((HC, AW), lambda i: (0, 0)),
            pl.BlockSpec((HC, AW), lambda i: (0, 0)),
            pl.BlockSpec((1, AW), lambda i: (0, 0)),
        ],
        out_specs=[
            pl.BlockSpec((rb, PW), lambda i: (i, 0)),
            pl.BlockSpec((rb, AW), lambda i: (i, 0)),
        ],
        out_shape=[
            jax.ShapeDtypeStruct((n, PW), jnp.float32),
            jax.ShapeDtypeStruct((n, AW), jnp.float32),
        ],
        interpret=interpret,
    )


def _make_combine(n, rb, act, interpret=False):
    def body(acc_ref, b_ref, krep_ref, o_ref):
        num = acc_ref[0, :, 0:HC] + acc_ref[1, :, 0:HC]
        den = acc_ref[0, :, HC:PW] + acc_ref[1, :, HC:PW]
        den_rep = jnp.dot(den, krep_ref[...], preferred_element_type=jnp.float32)
        out = num / (den_rep + 1e-16) + b_ref[...]
        if act:
            out = jnp.where(out > 0.0, out, jnp.exp(jnp.minimum(out, 0.0)) - 1.0)
        o_ref[...] = out

    return pl.pallas_call(
        body,
        grid=(n // rb,),
        in_specs=[
            pl.BlockSpec((NC, rb, PW), lambda i: (0, i, 0)),
            pl.BlockSpec((1, HC), lambda i: (0, 0)),
            pl.BlockSpec((AW, HC), lambda i: (0, 0)),
        ],
        out_specs=pl.BlockSpec((rb, HC), lambda i: (i, 0)),
        out_shape=jax.ShapeDtypeStruct((n, HC), jnp.float32),
        interpret=interpret,
    )


def _bcast_lane(v, j):
    dn = lax.GatherDimensionNumbers(
        offset_dims=(), collapsed_slice_dims=(0,), start_index_map=(0,))
    return lax.gather(
        v, jnp.full((16, 1), j, jnp.int32), dn, (1,),
        mode=lax.GatherScatterMode.PROMISE_IN_BOUNDS)


def _make_edge_kernel(n, e, chunk, unroll=4):
    tile_e = e // (NC * NS)
    npt = n // NS
    nch = tile_e // chunk
    pairs = nch // 2
    mesh = plsc.VectorSubcoreMesh(core_axis_name="c", subcore_axis_name="s")

    @functools.partial(
        pl.kernel,
        out_type=jax.ShapeDtypeStruct((NC, n, PW), jnp.float32),
        mesh=mesh,
        compiler_params=pltpu.CompilerParams(
            use_tc_tiling_on_sc=False, needs_layout_passes=False),
        scratch_types=[
            pltpu.VMEM((nch, chunk), jnp.int32),
            pltpu.VMEM((nch, chunk), jnp.int32),
            pltpu.VMEM((chunk, PW), jnp.float32),
            pltpu.VMEM((chunk, PW), jnp.float32),
            pltpu.VMEM((chunk, AW), jnp.float32),
            pltpu.VMEM((chunk, AW), jnp.float32),
            pltpu.VMEM((chunk, PW), jnp.float32),
            pltpu.VMEM_SHARED((n, PW), jnp.float32),
            pltpu.SemaphoreType.DMA,
            pltpu.SemaphoreType.DMA,
            pltpu.SemaphoreType.DMA,
            pltpu.SemaphoreType.DMA,
        ],
    )
    def edge_kernel(p_hbm, ad_hbm, src_hbm, dst_hbm, zeros_hbm, out_hbm,
                    sidx, didx, pbuf0, pbuf1, adbuf0, adbuf1, sbuf, acc,
                    semp0, semp1, sema0, sema1):
        c = lax.axis_index("c")
        s = lax.axis_index("s")
        w = c * NS + s

        pltpu.sync_copy(zeros_hbm.at[pl.ds(s * npt, npt)],
                        acc.at[pl.ds(s * npt, npt)])
        pltpu.sync_copy(src_hbm.at[pl.ds(w * nch, nch)], sidx)
        pltpu.sync_copy(dst_hbm.at[pl.ds(w * nch, nch)], didx)
        plsc.subcore_barrier()

        bufs = ((pbuf0, adbuf0, semp0, sema0), (pbuf1, adbuf1, semp1, sema1))

        def issue(k, b):
            pb, ab, sp, sa = bufs[b]
            dp = pltpu.async_copy(p_hbm.at[sidx.at[k]], pb, sp)
            da = pltpu.async_copy(ad_hbm.at[didx.at[k]], ab, sa)
            return dp, da

        def wait(k, b):
            pb, ab, sp, sa = bufs[b]
            pltpu.make_async_copy(p_hbm.at[sidx.at[k]], pb, sp).wait()
            pltpu.make_async_copy(ad_hbm.at[didx.at[k]], ab, sa).wait()

        def compute_scatter(k, b):
            pb, ab, _, _ = bufs[b]

            @plsc.parallel_loop(0, chunk, unroll=unroll)
            def edge_body(i):
                ev = pb[i, pl.ds(HC, AW)] + ab[i, :]
                ev = jnp.maximum(ev, ev * 0.2)
                z = jnp.exp(ev)
                sbuf[i, pl.ds(HC, AW)] = z
                for j in range(H):
                    zj = _bcast_lane(z, j)
                    sbuf[i, pl.ds(j * C, C)] = pb[i, pl.ds(j * C, C)] * zj

            pltpu.sync_copy(sbuf, acc.at[didx.at[k]], add=True)

        issue(0, 0)

        def pair_body(kk, carry):
            k0 = kk * 2
            issue(k0 + 1, 1)
            wait(k0, 0)
            compute_scatter(k0, 0)
            issue(k0 + 2, 0)
            wait(k0 + 1, 1)
            compute_scatter(k0 + 1, 1)
            return carry

        if nch % 2 == 1:
            lax.fori_loop(0, pairs, pair_body, 0)
            wait(nch - 1, 0)
            compute_scatter(nch - 1, 0)
        else:
            lax.fori_loop(0, pairs - 1, pair_body, 0)
            k0 = (pairs - 1) * 2
            issue(k0 + 1, 1)
            wait(k0, 0)
            compute_scatter(k0, 0)
            wait(k0 + 1, 1)
            compute_scatter(k0 + 1, 1)

        plsc.subcore_barrier()
        pltpu.sync_copy(acc.at[pl.ds(s * npt, npt)],
                        out_hbm.at[c, pl.ds(s * npt, npt)])

    return edge_kernel


def _att_mat(a):
    eye = jnp.eye(H, AW, dtype=jnp.float32)
    return (a[:, :, None] * eye[:, None, :]).reshape(HC, AW)


_PROJ = _make_proj(N, 1000)
_EDGE = _make_edge_kernel(N, E, CHUNK)
_COMBINE_ELU = _make_combine(N, 1000, act=True)
_COMBINE = _make_combine(N, 1000, act=False)


def kernel(x, edge_index, W1, a_src1, a_dst1, b1, W2, a_src2, a_dst2, b2):
    src2 = edge_index[0].reshape(E // CHUNK, CHUNK)
    dst2 = edge_index[1].reshape(E // CHUNK, CHUNK)
    zeros = jnp.zeros((N, PW), jnp.float32)
    krep = jnp.repeat(jnp.eye(AW, H, dtype=jnp.float32), C, axis=1)
    adb = jnp.concatenate(
        [jnp.zeros((1, H), jnp.float32),
         jnp.full((1, AW - H), -1e30, jnp.float32)], axis=1)

    p1, ad1 = _PROJ(x, W1, _att_mat(a_src1), _att_mat(a_dst1), adb)
    acc1 = _EDGE(p1, ad1, src2, dst2, zeros)
    g1 = _COMBINE_ELU(acc1, b1.reshape(1, HC), krep)

    p2, ad2 = _PROJ(g1, W2, _att_mat(a_src2), _att_mat(a_dst2), adb)
    acc2 = _EDGE(p2, ad2, src2, dst2, zeros)
    return _COMBINE(acc2, b2.reshape(1, HC), krep)


# speedup vs baseline: 167.2003x; 1.1516x over previous
import functools

import jax
import jax.numpy as jnp
from jax import lax
from jax.experimental import pallas as pl
from jax.experimental.pallas import tpu as pltpu
from jax.experimental.pallas import tpu_sc as plsc

N = 10000
E = 320000
D = 128
H = 8
C = 16
HC = H * C
AW = 16

NC = 2
NS = 16
CHUNK = 50


def _make_proj(n, rb, interpret=False):
    def body(x_ref, w_ref, as_ref, ad_ref, adb_ref, h_ref, aso_ref, ado_ref):
        h = jnp.dot(x_ref[...], w_ref[...], preferred_element_type=jnp.float32)
        h_ref[...] = h
        aso_ref[...] = jnp.dot(h, as_ref[...], preferred_element_type=jnp.float32)
        ado_ref[...] = (
            jnp.dot(h, ad_ref[...], preferred_element_type=jnp.float32)
            + adb_ref[...])

    return pl.pallas_call(
        body,
        grid=(n // rb,),
        in_specs=[
            pl.BlockSpec((rb, D), lambda i: (i, 0)),
            pl.BlockSpec((D, HC), lambda i: (0, 0)),
            pl.BlockSpec((HC, AW), lambda i: (0, 0)),
            pl.BlockSpec((HC, AW), lambda i: (0, 0)),
            pl.BlockSpec((1, AW), lambda i: (0, 0)),
        ],
        out_specs=[
            pl.BlockSpec((rb, HC), lambda i: (i, 0)),
            pl.BlockSpec((rb, AW), lambda i: (i, 0)),
            pl.BlockSpec((rb, AW), lambda i: (i, 0)),
        ],
        out_shape=[
            jax.ShapeDtypeStruct((n, HC), jnp.float32),
            jax.ShapeDtypeStruct((n, AW), jnp.float32),
            jax.ShapeDtypeStruct((n, AW), jnp.float32),
        ],
        interpret=interpret,
    )


def _make_combine(n, rb, act, interpret=False):
    def body(accn_ref, accd_ref, b_ref, krep_ref, o_ref):
        num = accn_ref[0] + accn_ref[1]
        den = accd_ref[0] + accd_ref[1]
        den_rep = jnp.dot(den, krep_ref[...], preferred_element_type=jnp.float32)
        out = num / (den_rep + 1e-16) + b_ref[...]
        if act:
            out = jnp.where(out > 0.0, out, jnp.exp(jnp.minimum(out, 0.0)) - 1.0)
        o_ref[...] = out

    return pl.pallas_call(
        body,
        grid=(n // rb,),
        in_specs=[
            pl.BlockSpec((NC, rb, HC), lambda i: (0, i, 0)),
            pl.BlockSpec((NC, rb, AW), lambda i: (0, i, 0)),
            pl.BlockSpec((1, HC), lambda i: (0, 0)),
            pl.BlockSpec((AW, HC), lambda i: (0, 0)),
        ],
        out_specs=pl.BlockSpec((rb, HC), lambda i: (i, 0)),
        out_shape=jax.ShapeDtypeStruct((n, HC), jnp.float32),
        interpret=interpret,
    )


def _bcast_lane(v, j):
    dn = lax.GatherDimensionNumbers(
        offset_dims=(), collapsed_slice_dims=(0,), start_index_map=(0,))
    return lax.gather(
        v, jnp.full((16, 1), j, jnp.int32), dn, (1,),
        mode=lax.GatherScatterMode.PROMISE_IN_BOUNDS)


def _make_edge_kernel(n, e, chunk, unroll=4):
    tile_e = e // (NC * NS)
    npt = n // NS
    nch = tile_e // chunk
    half = nch // 2
    pairs = half // 2
    assert nch % 2 == 0 and half % 2 == 0
    mesh = plsc.VectorSubcoreMesh(core_axis_name="c", subcore_axis_name="s")

    @functools.partial(
        pl.kernel,
        out_type=(jax.ShapeDtypeStruct((NC, n, HC), jnp.float32),
                  jax.ShapeDtypeStruct((NC, n, AW), jnp.float32)),
        mesh=mesh,
        compiler_params=pltpu.CompilerParams(
            use_tc_tiling_on_sc=False, needs_layout_passes=False),
        scratch_types=[
            pltpu.VMEM((half, chunk), jnp.int32),
            pltpu.VMEM((half, chunk), jnp.int32),
            pltpu.VMEM((chunk, HC), jnp.float32),
            pltpu.VMEM((chunk, HC), jnp.float32),
            pltpu.VMEM((chunk, AW), jnp.float32),
            pltpu.VMEM((chunk, AW), jnp.float32),
            pltpu.VMEM((chunk, AW), jnp.float32),
            pltpu.VMEM((chunk, AW), jnp.float32),
            pltpu.VMEM((chunk, HC), jnp.float32),
            pltpu.VMEM((chunk, AW), jnp.float32),
            pltpu.VMEM_SHARED((n, HC), jnp.float32),
            pltpu.VMEM_SHARED((n, AW), jnp.float32),
            pltpu.SemaphoreType.DMA,
            pltpu.SemaphoreType.DMA,
            pltpu.SemaphoreType.DMA,
            pltpu.SemaphoreType.DMA,
            pltpu.SemaphoreType.DMA,
        ],
    )
    def edge_kernel(h_hbm, as_hbm, ad_hbm, src_hbm, dst_hbm, zn_hbm, zd_hbm,
                    outn_hbm, outd_hbm,
                    sidx, didx, hbuf0, hbuf1, asbuf0, asbuf1, adbuf0, adbuf1,
                    sbn, sbd, accn, accd,
                    semh0, semh1, sema0, sema1, sems):
        c = lax.axis_index("c")
        s = lax.axis_index("s")
        w = c * NS + s

        pltpu.sync_copy(zn_hbm.at[pl.ds(s * npt, npt)],
                        accn.at[pl.ds(s * npt, npt)])
        pltpu.sync_copy(zd_hbm.at[pl.ds(s * npt, npt)],
                        accd.at[pl.ds(s * npt, npt)])
        plsc.subcore_barrier()

        bufs = ((hbuf0, asbuf0, adbuf0, semh0, sema0),
                (hbuf1, asbuf1, adbuf1, semh1, sema1))

        def issue(k, b):
            hb, ab, db, sh, sa = bufs[b]
            pltpu.async_copy(h_hbm.at[sidx.at[k]], hb, sh)
            pltpu.async_copy(as_hbm.at[sidx.at[k]], ab, sa)
            pltpu.async_copy(ad_hbm.at[didx.at[k]], db, sa)

        def wait_g(k, b):
            hb, ab, db, sh, sa = bufs[b]
            pltpu.make_async_copy(h_hbm.at[sidx.at[k]], hb, sh).wait()
            pltpu.make_async_copy(as_hbm.at[sidx.at[k]], ab, sa).wait()
            pltpu.make_async_copy(ad_hbm.at[didx.at[k]], db, sa).wait()

        def wait_s():
            pltpu.make_async_copy(sbn, accn.at[didx.at[0]], sems).wait()
            pltpu.make_async_copy(sbd, accd.at[didx.at[0]], sems).wait()

        def compute_scatter(k, b):
            hb, ab, db, _, _ = bufs[b]

            @plsc.parallel_loop(0, chunk, unroll=unroll)
            def edge_body(i):
                ev = ab[i, :] + db[i, :]
                ev = jnp.maximum(ev, ev * 0.2)
                z = jnp.exp(ev)
                sbd[i, :] = z
                for j in range(H):
                    zj = _bcast_lane(z, j)
                    sbn[i, pl.ds(j * C, C)] = hb[i, pl.ds(j * C, C)] * zj

            pltpu.async_copy(sbn, accn.at[didx.at[k]], sems, add=True)
            pltpu.async_copy(sbd, accd.at[didx.at[k]], sems, add=True)

        for hf in range(2):
            pltpu.sync_copy(src_hbm.at[pl.ds(w * nch + hf * half, half)], sidx)
            pltpu.sync_copy(dst_hbm.at[pl.ds(w * nch + hf * half, half)], didx)
            issue(0, 0)

            def pair_body(kk, carry):
                k0 = kk * 2
                issue(k0 + 1, 1)
                wait_g(k0, 0)

                @pl.when(kk > 0)
                def _():
                    wait_s()

                compute_scatter(k0, 0)
                issue(k0 + 2, 0)
                wait_g(k0 + 1, 1)
                wait_s()
                compute_scatter(k0 + 1, 1)
                return carry

            lax.fori_loop(0, pairs - 1, pair_body, 0)
            k0 = (pairs - 1) * 2
            issue(k0 + 1, 1)
            wait_g(k0, 0)
            wait_s()
            compute_scatter(k0, 0)
            wait_g(k0 + 1, 1)
            wait_s()
            compute_scatter(k0 + 1, 1)
            wait_s()

        plsc.subcore_barrier()
        pltpu.sync_copy(accn.at[pl.ds(s * npt, npt)],
                        outn_hbm.at[c, pl.ds(s * npt, npt)])
        pltpu.sync_copy(accd.at[pl.ds(s * npt, npt)],
                        outd_hbm.at[c, pl.ds(s * npt, npt)])

    return edge_kernel


def _att_mat(a):
    eye = jnp.eye(H, AW, dtype=jnp.float32)
    return (a[:, :, None] * eye[:, None, :]).reshape(HC, AW)


_PROJ = _make_proj(N, 1000)
_EDGE = _make_edge_kernel(N, E, CHUNK)
_COMBINE_ELU = _make_combine(N, 1000, act=True)
_COMBINE = _make_combine(N, 1000, act=False)


def kernel(x, edge_index, W1, a_src1, a_dst1, b1, W2, a_src2, a_dst2, b2):
    src2 = edge_index[0].reshape(E // CHUNK, CHUNK)
    dst2 = edge_index[1].reshape(E // CHUNK, CHUNK)
    zn = jnp.zeros((N, HC), jnp.float32)
    zd = jnp.zeros((N, AW), jnp.float32)
    krep = jnp.repeat(jnp.eye(AW, H, dtype=jnp.float32), C, axis=1)
    adb = jnp.concatenate(
        [jnp.zeros((1, H), jnp.float32),
         jnp.full((1, AW - H), -1e30, jnp.float32)], axis=1)

    h1, as1, ad1 = _PROJ(x, W1, _att_mat(a_src1), _att_mat(a_dst1), adb)
    an1, ad1acc = _EDGE(h1, as1, ad1, src2, dst2, zn, zd)
    g1 = _COMBINE_ELU(an1, ad1acc, b1.reshape(1, HC), krep)

    h2, as2, ad2 = _PROJ(g1, W2, _att_mat(a_src2), _att_mat(a_dst2), adb)
    an2, ad2acc = _EDGE(h2, as2, ad2, src2, dst2, zn, zd)
    return _COMBINE(an2, ad2acc, b2.reshape(1, HC), krep)


# speedup vs baseline: 170.7499x; 1.0212x over previous
import functools

import jax
import jax.numpy as jnp
from jax import lax
from jax.experimental import pallas as pl
from jax.experimental.pallas import tpu as pltpu
from jax.experimental.pallas import tpu_sc as plsc

N = 10000
E = 320000
D = 128
H = 8
C = 16
HC = H * C
AW = 16

NC = 2
NS = 16
CHUNK = 50


def _make_proj(n, rb, interpret=False):
    def body(x_ref, w_ref, as_ref, ad_ref, adb_ref, h_ref, aso_ref, ado_ref):
        h = jnp.dot(x_ref[...], w_ref[...], preferred_element_type=jnp.float32)
        h_ref[...] = h
        aso_ref[...] = jnp.dot(h, as_ref[...], preferred_element_type=jnp.float32)
        ado_ref[...] = (
            jnp.dot(h, ad_ref[...], preferred_element_type=jnp.float32)
            + adb_ref[...])

    return pl.pallas_call(
        body,
        grid=(n // rb,),
        in_specs=[
            pl.BlockSpec((rb, D), lambda i: (i, 0)),
            pl.BlockSpec((D, HC), lambda i: (0, 0)),
            pl.BlockSpec((HC, AW), lambda i: (0, 0)),
            pl.BlockSpec((HC, AW), lambda i: (0, 0)),
            pl.BlockSpec((1, AW), lambda i: (0, 0)),
        ],
        out_specs=[
            pl.BlockSpec((rb, HC), lambda i: (i, 0)),
            pl.BlockSpec((rb, AW), lambda i: (i, 0)),
            pl.BlockSpec((rb, AW), lambda i: (i, 0)),
        ],
        out_shape=[
            jax.ShapeDtypeStruct((n, HC), jnp.float32),
            jax.ShapeDtypeStruct((n, AW), jnp.float32),
            jax.ShapeDtypeStruct((n, AW), jnp.float32),
        ],
        interpret=interpret,
    )


def _make_combine(n, rb, act, interpret=False):
    def body(accn_ref, accd_ref, b_ref, krep_ref, o_ref):
        num = accn_ref[0] + accn_ref[1]
        den = accd_ref[0] + accd_ref[1]
        den_rep = jnp.dot(den, krep_ref[...], preferred_element_type=jnp.float32)
        out = num / (den_rep + 1e-16) + b_ref[...]
        if act:
            out = jnp.where(out > 0.0, out, jnp.exp(jnp.minimum(out, 0.0)) - 1.0)
        o_ref[...] = out

    return pl.pallas_call(
        body,
        grid=(n // rb,),
        in_specs=[
            pl.BlockSpec((NC, rb, HC), lambda i: (0, i, 0)),
            pl.BlockSpec((NC, rb, AW), lambda i: (0, i, 0)),
            pl.BlockSpec((1, HC), lambda i: (0, 0)),
            pl.BlockSpec((AW, HC), lambda i: (0, 0)),
        ],
        out_specs=pl.BlockSpec((rb, HC), lambda i: (i, 0)),
        out_shape=jax.ShapeDtypeStruct((n, HC), jnp.float32),
        interpret=interpret,
    )


def _bcast_lane(v, j):
    dn = lax.GatherDimensionNumbers(
        offset_dims=(), collapsed_slice_dims=(0,), start_index_map=(0,))
    return lax.gather(
        v, jnp.full((16, 1), j, jnp.int32), dn, (1,),
        mode=lax.GatherScatterMode.PROMISE_IN_BOUNDS)


def _make_edge_kernel(n, e, chunk, unroll=4):
    tile_e = e // (NC * NS)
    npt = n // NS
    nch = tile_e // chunk
    half = nch // 2
    pairs = half // 2
    assert nch % 2 == 0 and half % 2 == 0
    mesh = plsc.VectorSubcoreMesh(core_axis_name="c", subcore_axis_name="s")

    @functools.partial(
        pl.kernel,
        out_type=(jax.ShapeDtypeStruct((NC, n, HC), jnp.float32),
                  jax.ShapeDtypeStruct((NC, n, AW), jnp.float32)),
        mesh=mesh,
        compiler_params=pltpu.CompilerParams(
            use_tc_tiling_on_sc=False, needs_layout_passes=False),
        scratch_types=[
            pltpu.VMEM((half, chunk), jnp.int32),
            pltpu.VMEM((half, chunk), jnp.int32),
            pltpu.VMEM((chunk, HC), jnp.float32),
            pltpu.VMEM((chunk, HC), jnp.float32),
            pltpu.VMEM((chunk, AW), jnp.float32),
            pltpu.VMEM((chunk, AW), jnp.float32),
            pltpu.VMEM((chunk, AW), jnp.float32),
            pltpu.VMEM((chunk, AW), jnp.float32),
            pltpu.VMEM((chunk, HC), jnp.float32),
            pltpu.VMEM((chunk, AW), jnp.float32),
            pltpu.VMEM_SHARED((n, HC), jnp.float32),
            pltpu.VMEM_SHARED((n, AW), jnp.float32),
            pltpu.SemaphoreType.DMA,
            pltpu.SemaphoreType.DMA,
            pltpu.SemaphoreType.DMA,
            pltpu.SemaphoreType.DMA,
            pltpu.SemaphoreType.DMA,
        ],
    )
    def edge_kernel(h_hbm, as_hbm, ad_hbm, ei_hbm, zn_hbm, zd_hbm,
                    outn_hbm, outd_hbm,
                    sidx, didx, hbuf0, hbuf1, asbuf0, asbuf1, adbuf0, adbuf1,
                    sbn, sbd, accn, accd,
                    semh0, semh1, sema0, sema1, sems):
        c = lax.axis_index("c")
        s = lax.axis_index("s")
        w = c * NS + s

        pltpu.sync_copy(zn_hbm.at[pl.ds(s * npt, npt)],
                        accn.at[pl.ds(s * npt, npt)])
        pltpu.sync_copy(zd_hbm.at[pl.ds(s * npt, npt)],
                        accd.at[pl.ds(s * npt, npt)])
        plsc.subcore_barrier()

        bufs = ((hbuf0, asbuf0, adbuf0, semh0, sema0),
                (hbuf1, asbuf1, adbuf1, semh1, sema1))

        def sl(t, k):
            return t.at[k]

        def issue(k, b):
            hb, ab, db, sh, sa = bufs[b]
            pltpu.async_copy(h_hbm.at[sl(sidx, k)], hb, sh)
            pltpu.async_copy(as_hbm.at[sl(sidx, k)], ab, sa)
            pltpu.async_copy(ad_hbm.at[sl(didx, k)], db, sa)

        def wait_g(k, b):
            hb, ab, db, sh, sa = bufs[b]
            pltpu.make_async_copy(h_hbm.at[sl(sidx, k)], hb, sh).wait()
            pltpu.make_async_copy(as_hbm.at[sl(sidx, k)], ab, sa).wait()
            pltpu.make_async_copy(ad_hbm.at[sl(didx, k)], db, sa).wait()

        def wait_s():
            pltpu.make_async_copy(sbn, accn.at[sl(didx, 0)], sems).wait()
            pltpu.make_async_copy(sbd, accd.at[sl(didx, 0)], sems).wait()

        def compute_scatter(k, b):
            hb, ab, db, _, _ = bufs[b]

            @plsc.parallel_loop(0, chunk, unroll=unroll)
            def edge_body(i):
                ev = ab[i, :] + db[i, :]
                ev = jnp.maximum(ev, ev * 0.2)
                z = jnp.exp(ev)
                sbd[i, :] = z
                for j in range(H):
                    zj = _bcast_lane(z, j)
                    sbn[i, pl.ds(j * C, C)] = hb[i, pl.ds(j * C, C)] * zj

            pltpu.async_copy(sbn, accn.at[sl(didx, k)], sems, add=True)
            pltpu.async_copy(sbd, accd.at[sl(didx, k)], sems, add=True)

        for hf in range(2):
            pltpu.sync_copy(ei_hbm.at[0, pl.ds(w * nch + hf * half, half)], sidx)
            pltpu.sync_copy(ei_hbm.at[1, pl.ds(w * nch + hf * half, half)], didx)
            issue(0, 0)

            def pair_body(kk, carry):
                k0 = kk * 2
                issue(k0 + 1, 1)
                wait_g(k0, 0)

                @pl.when(kk > 0)
                def _():
                    wait_s()

                compute_scatter(k0, 0)
                issue(k0 + 2, 0)
                wait_g(k0 + 1, 1)
                wait_s()
                compute_scatter(k0 + 1, 1)
                return carry

            lax.fori_loop(0, pairs - 1, pair_body, 0)
            k0 = (pairs - 1) * 2
            issue(k0 + 1, 1)
            wait_g(k0, 0)
            wait_s()
            compute_scatter(k0, 0)
            wait_g(k0 + 1, 1)
            wait_s()
            compute_scatter(k0 + 1, 1)
            wait_s()

        plsc.subcore_barrier()
        pltpu.sync_copy(accn.at[pl.ds(s * npt, npt)],
                        outn_hbm.at[c, pl.ds(s * npt, npt)])
        pltpu.sync_copy(accd.at[pl.ds(s * npt, npt)],
                        outd_hbm.at[c, pl.ds(s * npt, npt)])

    return edge_kernel


def _att_mat(a):
    eye = jnp.eye(H, AW, dtype=jnp.float32)
    return (a[:, :, None] * eye[:, None, :]).reshape(HC, AW)


_PROJ = _make_proj(N, 1000)
_EDGE = _make_edge_kernel(N, E, CHUNK)
_COMBINE_ELU = _make_combine(N, 1000, act=True)
_COMBINE = _make_combine(N, 1000, act=False)


def kernel(x, edge_index, W1, a_src1, a_dst1, b1, W2, a_src2, a_dst2, b2):
    zn = jnp.zeros((N, HC), jnp.float32)
    zd = jnp.zeros((N, AW), jnp.float32)
    krep = jnp.repeat(jnp.eye(AW, H, dtype=jnp.float32), C, axis=1)
    adb = jnp.concatenate(
        [jnp.zeros((1, H), jnp.float32),
         jnp.full((1, AW - H), -1e30, jnp.float32)], axis=1)

    ei3 = edge_index.reshape(2, E // CHUNK, CHUNK)
    h1, as1, ad1 = _PROJ(x, W1, _att_mat(a_src1), _att_mat(a_dst1), adb)
    an1, ad1acc = _EDGE(h1, as1, ad1, ei3, zn, zd)
    g1 = _COMBINE_ELU(an1, ad1acc, b1.reshape(1, HC), krep)

    h2, as2, ad2 = _PROJ(g1, W2, _att_mat(a_src2), _att_mat(a_dst2), adb)
    an2, ad2acc = _EDGE(h2, as2, ad2, ei3, zn, zd)
    return _COMBINE(an2, ad2acc, b2.reshape(1, HC), krep)


# speedup vs baseline: 174.1729x; 1.0200x over previous
import functools

import jax
import jax.numpy as jnp
from jax import lax
from jax.experimental import pallas as pl
from jax.experimental.pallas import tpu as pltpu
from jax.experimental.pallas import tpu_sc as plsc

N = 10000
E = 320000
D = 128
H = 8
C = 16
HC = H * C
AW = 16

NC = 2
NS = 16
CHUNK = 50


def _make_proj(n, rb, interpret=False):
    def body(x_ref, w_ref, as_ref, ad_ref, adb_ref, h_ref, aso_ref, ado_ref):
        h = jnp.dot(x_ref[...], w_ref[...], preferred_element_type=jnp.float32)
        h_ref[...] = h
        aso_ref[...] = jnp.dot(h, as_ref[...], preferred_element_type=jnp.float32)
        ado_ref[...] = (
            jnp.dot(h, ad_ref[...], preferred_element_type=jnp.float32)
            + adb_ref[...])

    return pl.pallas_call(
        body,
        grid=(n // rb,),
        in_specs=[
            pl.BlockSpec((rb, D), lambda i: (i, 0)),
            pl.BlockSpec((D, HC), lambda i: (0, 0)),
            pl.BlockSpec((HC, AW), lambda i: (0, 0)),
            pl.BlockSpec((HC, AW), lambda i: (0, 0)),
            pl.BlockSpec((1, AW), lambda i: (0, 0)),
        ],
        out_specs=[
            pl.BlockSpec((rb, HC), lambda i: (i, 0)),
            pl.BlockSpec((rb, AW), lambda i: (i, 0)),
            pl.BlockSpec((rb, AW), lambda i: (i, 0)),
        ],
        out_shape=[
            jax.ShapeDtypeStruct((n, HC), jnp.float32),
            jax.ShapeDtypeStruct((n, AW), jnp.float32),
            jax.ShapeDtypeStruct((n, AW), jnp.float32),
        ],
        interpret=interpret,
    )


def _make_combine(n, rb, act, interpret=False):
    def body(accn_ref, accd_ref, b_ref, krep_ref, o_ref):
        num = accn_ref[0] + accn_ref[1]
        den = accd_ref[0] + accd_ref[1]
        den_rep = jnp.dot(den, krep_ref[...], preferred_element_type=jnp.float32)
        out = num / (den_rep + 1e-16) + b_ref[...]
        if act:
            out = jnp.where(out > 0.0, out, jnp.exp(jnp.minimum(out, 0.0)) - 1.0)
        o_ref[...] = out

    return pl.pallas_call(
        body,
        grid=(n // rb,),
        in_specs=[
            pl.BlockSpec((NC, rb, HC), lambda i: (0, i, 0)),
            pl.BlockSpec((NC, rb, AW), lambda i: (0, i, 0)),
            pl.BlockSpec((1, HC), lambda i: (0, 0)),
            pl.BlockSpec((AW, HC), lambda i: (0, 0)),
        ],
        out_specs=pl.BlockSpec((rb, HC), lambda i: (i, 0)),
        out_shape=jax.ShapeDtypeStruct((n, HC), jnp.float32),
        interpret=interpret,
    )


def _make_combine_proj(n, rb, interpret=False):
    def body(accn_ref, accd_ref, b_ref, krep_ref, w_ref, as_ref, ad_ref,
             adb_ref, h_ref, aso_ref, ado_ref):
        num = accn_ref[0] + accn_ref[1]
        den = accd_ref[0] + accd_ref[1]
        den_rep = jnp.dot(den, krep_ref[...], preferred_element_type=jnp.float32)
        g = num / (den_rep + 1e-16) + b_ref[...]
        g = jnp.where(g > 0.0, g, jnp.exp(jnp.minimum(g, 0.0)) - 1.0)
        h = jnp.dot(g, w_ref[...], preferred_element_type=jnp.float32)
        h_ref[...] = h
        aso_ref[...] = jnp.dot(h, as_ref[...], preferred_element_type=jnp.float32)
        ado_ref[...] = (
            jnp.dot(h, ad_ref[...], preferred_element_type=jnp.float32)
            + adb_ref[...])

    return pl.pallas_call(
        body,
        grid=(n // rb,),
        in_specs=[
            pl.BlockSpec((NC, rb, HC), lambda i: (0, i, 0)),
            pl.BlockSpec((NC, rb, AW), lambda i: (0, i, 0)),
            pl.BlockSpec((1, HC), lambda i: (0, 0)),
            pl.BlockSpec((AW, HC), lambda i: (0, 0)),
            pl.BlockSpec((HC, HC), lambda i: (0, 0)),
            pl.BlockSpec((HC, AW), lambda i: (0, 0)),
            pl.BlockSpec((HC, AW), lambda i: (0, 0)),
            pl.BlockSpec((1, AW), lambda i: (0, 0)),
        ],
        out_specs=[
            pl.BlockSpec((rb, HC), lambda i: (i, 0)),
            pl.BlockSpec((rb, AW), lambda i: (i, 0)),
            pl.BlockSpec((rb, AW), lambda i: (i, 0)),
        ],
        out_shape=[
            jax.ShapeDtypeStruct((n, HC), jnp.float32),
            jax.ShapeDtypeStruct((n, AW), jnp.float32),
            jax.ShapeDtypeStruct((n, AW), jnp.float32),
        ],
        interpret=interpret,
    )


def _bcast_lane(v, j):
    dn = lax.GatherDimensionNumbers(
        offset_dims=(), collapsed_slice_dims=(0,), start_index_map=(0,))
    return lax.gather(
        v, jnp.full((16, 1), j, jnp.int32), dn, (1,),
        mode=lax.GatherScatterMode.PROMISE_IN_BOUNDS)


def _make_edge_kernel(n, e, chunk, unroll=4):
    tile_e = e // (NC * NS)
    npt = n // NS
    nch = tile_e // chunk
    half = nch // 2
    pairs = half // 2
    assert nch % 2 == 0 and half % 2 == 0
    mesh = plsc.VectorSubcoreMesh(core_axis_name="c", subcore_axis_name="s")

    @functools.partial(
        pl.kernel,
        out_type=(jax.ShapeDtypeStruct((NC, n, HC), jnp.float32),
                  jax.ShapeDtypeStruct((NC, n, AW), jnp.float32)),
        mesh=mesh,
        compiler_params=pltpu.CompilerParams(
            use_tc_tiling_on_sc=False, needs_layout_passes=False),
        scratch_types=[
            pltpu.VMEM((half, chunk), jnp.int32),
            pltpu.VMEM((half, chunk), jnp.int32),
            pltpu.VMEM((chunk, HC), jnp.float32),
            pltpu.VMEM((chunk, HC), jnp.float32),
            pltpu.VMEM((chunk, AW), jnp.float32),
            pltpu.VMEM((chunk, AW), jnp.float32),
            pltpu.VMEM((chunk, AW), jnp.float32),
            pltpu.VMEM((chunk, AW), jnp.float32),
            pltpu.VMEM((chunk, HC), jnp.float32),
            pltpu.VMEM((chunk, AW), jnp.float32),
            pltpu.VMEM_SHARED((n, HC), jnp.float32),
            pltpu.VMEM_SHARED((n, AW), jnp.float32),
            pltpu.SemaphoreType.DMA,
            pltpu.SemaphoreType.DMA,
            pltpu.SemaphoreType.DMA,
            pltpu.SemaphoreType.DMA,
            pltpu.SemaphoreType.DMA,
        ],
    )
    def edge_kernel(h_hbm, as_hbm, ad_hbm, ei_hbm, zn_hbm, zd_hbm,
                    outn_hbm, outd_hbm,
                    sidx, didx, hbuf0, hbuf1, asbuf0, asbuf1, adbuf0, adbuf1,
                    sbn, sbd, accn, accd,
                    semh0, semh1, sema0, sema1, sems):
        c = lax.axis_index("c")
        s = lax.axis_index("s")
        w = c * NS + s

        pltpu.sync_copy(zn_hbm.at[pl.ds(s * npt, npt)],
                        accn.at[pl.ds(s * npt, npt)])
        pltpu.sync_copy(zd_hbm.at[pl.ds(s * npt, npt)],
                        accd.at[pl.ds(s * npt, npt)])
        plsc.subcore_barrier()

        bufs = ((hbuf0, asbuf0, adbuf0, semh0, sema0),
                (hbuf1, asbuf1, adbuf1, semh1, sema1))

        def sl(t, k):
            return t.at[k]

        def issue(k, b):
            hb, ab, db, sh, sa = bufs[b]
            pltpu.async_copy(h_hbm.at[sl(sidx, k)], hb, sh)
            pltpu.async_copy(as_hbm.at[sl(sidx, k)], ab, sa)
            pltpu.async_copy(ad_hbm.at[sl(didx, k)], db, sa)

        def wait_g(k, b):
            hb, ab, db, sh, sa = bufs[b]
            pltpu.make_async_copy(h_hbm.at[sl(sidx, k)], hb, sh).wait()
            pltpu.make_async_copy(as_hbm.at[sl(sidx, k)], ab, sa).wait()
            pltpu.make_async_copy(ad_hbm.at[sl(didx, k)], db, sa).wait()

        def wait_s():
            pltpu.make_async_copy(sbn, accn.at[sl(didx, 0)], sems).wait()
            pltpu.make_async_copy(sbd, accd.at[sl(didx, 0)], sems).wait()

        def compute_scatter(k, b):
            hb, ab, db, _, _ = bufs[b]

            @plsc.parallel_loop(0, chunk, unroll=unroll)
            def edge_body(i):
                ev = ab[i, :] + db[i, :]
                ev = jnp.maximum(ev, ev * 0.2)
                z = jnp.exp(ev)
                sbd[i, :] = z
                for j in range(H):
                    zj = _bcast_lane(z, j)
                    sbn[i, pl.ds(j * C, C)] = hb[i, pl.ds(j * C, C)] * zj

            pltpu.async_copy(sbn, accn.at[sl(didx, k)], sems, add=True)
            pltpu.async_copy(sbd, accd.at[sl(didx, k)], sems, add=True)

        for hf in range(2):
            pltpu.sync_copy(ei_hbm.at[0, pl.ds(w * nch + hf * half, half)], sidx)
            pltpu.sync_copy(ei_hbm.at[1, pl.ds(w * nch + hf * half, half)], didx)
            issue(0, 0)

            def pair_body(kk, carry):
                k0 = kk * 2
                issue(k0 + 1, 1)
                wait_g(k0, 0)

                @pl.when(kk > 0)
                def _():
                    wait_s()

                compute_scatter(k0, 0)
                issue(k0 + 2, 0)
                wait_g(k0 + 1, 1)
                wait_s()
                compute_scatter(k0 + 1, 1)
                return carry

            lax.fori_loop(0, pairs - 1, pair_body, 0)
            k0 = (pairs - 1) * 2
            issue(k0 + 1, 1)
            wait_g(k0, 0)
            wait_s()
            compute_scatter(k0, 0)
            wait_g(k0 + 1, 1)
            wait_s()
            compute_scatter(k0 + 1, 1)
            wait_s()

        plsc.subcore_barrier()
        pltpu.sync_copy(accn.at[pl.ds(s * npt, npt)],
                        outn_hbm.at[c, pl.ds(s * npt, npt)])
        pltpu.sync_copy(accd.at[pl.ds(s * npt, npt)],
                        outd_hbm.at[c, pl.ds(s * npt, npt)])

    return edge_kernel


def _att_mat(a):
    eye = jnp.eye(H, AW, dtype=jnp.float32)
    return (a[:, :, None] * eye[:, None, :]).reshape(HC, AW)


_PROJ = _make_proj(N, 1000)
_EDGE = _make_edge_kernel(N, E, CHUNK)
_COMBINE_PROJ = _make_combine_proj(N, 1000)
_COMBINE = _make_combine(N, 1000, act=False)


def kernel(x, edge_index, W1, a_src1, a_dst1, b1, W2, a_src2, a_dst2, b2):
    zn = jnp.zeros((N, HC), jnp.float32)
    zd = jnp.zeros((N, AW), jnp.float32)
    krep = jnp.repeat(jnp.eye(AW, H, dtype=jnp.float32), C, axis=1)
    adb = jnp.concatenate(
        [jnp.zeros((1, H), jnp.float32),
         jnp.full((1, AW - H), -1e30, jnp.float32)], axis=1)

    ei3 = edge_index.reshape(2, E // CHUNK, CHUNK)
    h1, as1, ad1 = _PROJ(x, W1, _att_mat(a_src1), _att_mat(a_dst1), adb)
    an1, ad1acc = _EDGE(h1, as1, ad1, ei3, zn, zd)
    h2, as2, ad2 = _COMBINE_PROJ(
        an1, ad1acc, b1.reshape(1, HC), krep, W2,
        _att_mat(a_src2), _att_mat(a_dst2), adb)
    an2, ad2acc = _EDGE(h2, as2, ad2, ei3, zn, zd)
    return _COMBINE(an2, ad2acc, b2.reshape(1, HC), krep)
